# Initial kernel scaffold; baseline (speedup 1.0000x reference)
import jax
import jax.numpy as jnp
from jax.experimental import pallas as pl


def kernel(x, edge_index, batch, W1, b1, W2, b2, Wg, att_src, att_dst, bg, Wl1, bl1, Wl2, bl2):
    raise NotImplementedError("write your pallas kernel here")


# speedup vs baseline: 1.0012x; 1.0012x over previous
import jax
import jax.numpy as jnp
from jax.experimental import pallas as pl
from jax.experimental.pallas import tpu as pltpu

N = 10000
E = 320000
F_IN = 128
HID = 128
HEADS = 2
NCLS = 2
NGRAPH = 64


def _pool_mlp_body(h_ref, batch_ref, wl1_ref, bl1_ref, wl2_ref, bl2_ref,
                   o_ref, mx_ref):
    h = h_ref[...]
    batch = batch_ref[...]
    gids = jax.lax.broadcasted_iota(jnp.int32, (N, NGRAPH), 1)
    onehot = (batch[:, None] == gids).astype(jnp.float32)
    counts = jnp.sum(onehot, axis=0)
    meansum = jax.lax.dot_general(onehot, h, (((0,), (0,)), ((), ())),
                                  preferred_element_type=jnp.float32)
    mean = meansum / jnp.maximum(counts, 1.0)[:, None]

    def body(g, _):
        m = jnp.where(batch[:, None] == g, h, -jnp.inf)
        row = jnp.max(m, axis=0)
        row = jnp.where(jnp.isfinite(row), row, 0.0)
        mx_ref[pl.ds(g, 1), :] = row[None, :]
        return 0

    jax.lax.fori_loop(0, NGRAPH, body, 0)
    g = mean + mx_ref[...]
    g = jax.nn.relu(jnp.dot(g, wl1_ref[...],
                            preferred_element_type=jnp.float32) + bl1_ref[...])
    o_ref[...] = jnp.dot(g, wl2_ref[...],
                         preferred_element_type=jnp.float32) + bl2_ref[...]


def _pool_mlp(h, batch, Wl1, bl1, Wl2, bl2):
    return pl.pallas_call(
        _pool_mlp_body,
        out_shape=jax.ShapeDtypeStruct((NGRAPH, NCLS), jnp.float32),
        scratch_shapes=[pltpu.VMEM((NGRAPH, HID), jnp.float32)],
    )(h, batch, Wl1, bl1, Wl2, bl2)


def kernel(x, edge_index, batch, W1, b1, W2, b2, Wg, att_src, att_dst, bg, Wl1, bl1, Wl2, bl2):
    loop = jnp.arange(N, dtype=edge_index.dtype)
    src = jnp.concatenate([edge_index[0], loop])
    dst = jnp.concatenate([edge_index[1], loop])
    deg = jax.ops.segment_sum(jnp.ones_like(src, jnp.float32), dst, num_segments=N)
    dis = jnp.where(deg > 0, 1.0 / jnp.sqrt(deg), 0.0)
    norm = dis[src] * dis[dst]

    def gcn(xin, W, b):
        h = xin @ W
        msg = h[src] * norm[:, None]
        return jax.ops.segment_sum(msg, dst, num_segments=N) + b

    h = jax.nn.relu(gcn(x, W1, b1))
    h = jax.nn.relu(gcn(h, W2, b2))

    hg = (h @ Wg).reshape(N, HEADS, HID)
    a_src = jnp.sum(hg * att_src[None], axis=-1)
    a_dst = jnp.sum(hg * att_dst[None], axis=-1)
    e = jax.nn.leaky_relu(a_src[src] + a_dst[dst], negative_slope=0.2)
    m = jax.ops.segment_max(e, dst, num_segments=N)
    m = jnp.where(jnp.isfinite(m), m, 0.0)
    ex = jnp.exp(e - m[dst])
    s = jax.ops.segment_sum(ex, dst, num_segments=N)
    alpha = ex / (s[dst] + 1e-16)
    msg = hg[src] * alpha[:, :, None]
    out = jax.ops.segment_sum(msg, dst, num_segments=N)
    h = jax.nn.relu(jnp.mean(out, axis=1) + bg)

    return _pool_mlp(h, batch, Wl1, bl1, Wl2, bl2)


# speedup vs baseline: 24.1171x; 24.0883x over previous
import functools

import jax
import jax.numpy as jnp
from jax import lax
from jax.experimental import pallas as pl
from jax.experimental.pallas import tpu as pltpu
from jax.experimental.pallas import tpu_sc as plsc

N = 10000
E = 320000
F_IN = 128
HID = 128
HEADS = 2
NCLS = 2
NGRAPH = 64

NC = 2
NS = 16
NW = NC * NS
L = 16

EW = E // NW
K = 128
NCH = 80
NBLK = NCH // 8
EWP = NCH * K

NE = N + 8
RPT = 624
RTAIL = NS * RPT
RREM = NE - RTAIL


def _mesh():
    return plsc.VectorSubcoreMesh(core_axis_name="c", subcore_axis_name="s")


def _wid():
    return lax.axis_index("s") * NC + lax.axis_index("c")


def _deg_body(dst_hbm, out_hbm, deg_v, idx_v):
    w = _wid()
    zero = jnp.zeros((L,), jnp.float32)

    def zbody(i, _):
        deg_v[pl.ds(i * L, L)] = zero
        return 0
    lax.fori_loop(0, N // L, zbody, 0)

    pltpu.sync_copy(dst_hbm.at[w], idx_v)
    ones = jnp.ones((L,), jnp.float32)

    def body(i, _):
        d = idx_v[pl.ds(i * L, L)]
        plsc.addupdate_scatter(deg_v, [d], ones)
        return 0
    lax.fori_loop(0, EW // L, body, 0)

    pltpu.sync_copy(deg_v, out_hbm.at[w])


_deg_kernel = functools.partial(
    pl.kernel,
    out_type=jax.ShapeDtypeStruct((NW, N), jnp.float32),
    mesh=_mesh(),
    compiler_params=pltpu.CompilerParams(needs_layout_passes=False),
    scratch_types=[
        pltpu.VMEM((N,), jnp.float32),
        pltpu.VMEM((EW,), jnp.int32),
    ],
)(_deg_body)


def _gcn_body(hs_hbm, src_hbm, dst_hbm, zeros_hbm, out_hbm,
              srcblk, dstblk, buf, semi, semg0, semg1, acc):
    c = lax.axis_index("c")
    s = lax.axis_index("s")
    w = _wid()
    rs = s * RPT
    pltpu.sync_copy(zeros_hbm.at[pl.ds(rs, RPT)], acc.at[pl.ds(rs, RPT)])

    @pl.when(s == NS - 1)
    def _():
        pltpu.sync_copy(zeros_hbm.at[pl.ds(RTAIL, RREM)],
                        acc.at[pl.ds(RTAIL, RREM)])
    plsc.subcore_barrier()

    semg = (semg0, semg1)

    def issue_blk(b):
        sl = jnp.bitwise_and(b, 1)
        pltpu.async_copy(src_hbm.at[w * NBLK + b], srcblk.at[sl], semi)
        pltpu.async_copy(dst_hbm.at[w * NBLK + b], dstblk.at[sl], semi)

    def wait_blk(b):
        sl = jnp.bitwise_and(b, 1)
        pltpu.make_async_copy(src_hbm.at[w * NBLK + b], srcblk.at[sl],
                              semi).wait()
        pltpu.make_async_copy(dst_hbm.at[w * NBLK + b], dstblk.at[sl],
                              semi).wait()

    def issue_gather(cc, p):
        idx = srcblk.at[jnp.bitwise_and(lax.shift_right_logical(cc, 3), 1),
                        jnp.bitwise_and(cc, 7)]
        pltpu.async_copy(hs_hbm.at[idx], buf.at[p], semg[p])

    def wait_gather(cc, p):
        idx = srcblk.at[jnp.bitwise_and(lax.shift_right_logical(cc, 3), 1),
                        jnp.bitwise_and(cc, 7)]
        pltpu.make_async_copy(hs_hbm.at[idx], buf.at[p], semg[p]).wait()

    issue_blk(0)
    wait_blk(0)
    issue_blk(1)
    issue_gather(0, 0)

    def chunk(cc, p):
        c1 = cc + 1

        @pl.when(c1 < NCH)
        def _():
            @pl.when(jnp.bitwise_and(c1, 7) == 0)
            def _():
                wait_blk(lax.shift_right_logical(c1, 3))
            issue_gather(c1, 1 - p)

        wait_gather(cc, p)
        didx = dstblk.at[jnp.bitwise_and(lax.shift_right_logical(cc, 3), 1),
                         jnp.bitwise_and(cc, 7)]
        pltpu.sync_copy(buf.at[p], acc.at[didx], add=True)

        @pl.when(jnp.logical_and(jnp.bitwise_and(c1, 7) == 0,
                                 c1 + 8 < NCH))
        def _():
            issue_blk(lax.shift_right_logical(c1, 3) + 1)

    def super_body(t, _):
        chunk(2 * t, 0)
        chunk(2 * t + 1, 1)
        return 0
    lax.fori_loop(0, NCH // 2, super_body, 0)

    plsc.subcore_barrier()
    pltpu.sync_copy(acc.at[pl.ds(rs, RPT)],
                    out_hbm.at[pl.ds(c * NE + rs, RPT)])

    @pl.when(s == NS - 1)
    def _():
        pltpu.sync_copy(acc.at[pl.ds(RTAIL, RREM)],
                        out_hbm.at[pl.ds(c * NE + RTAIL, RREM)])


_gcn_kernel = functools.partial(
    pl.kernel,
    out_type=jax.ShapeDtypeStruct((NC * NE, HID), jnp.float32),
    mesh=_mesh(),
    compiler_params=pltpu.CompilerParams(needs_layout_passes=False),
    scratch_types=[
        pltpu.VMEM((2, 8, K), jnp.int32),
        pltpu.VMEM((2, 8, K), jnp.int32),
        pltpu.VMEM((2, K, HID), jnp.float32),
        pltpu.SemaphoreType.DMA,
        pltpu.SemaphoreType.DMA,
        pltpu.SemaphoreType.DMA,
        pltpu.VMEM_SHARED((NE, HID), jnp.float32),
    ],
)(_gcn_body)


def _gata_body(asrc_hbm, adst_hbm, src_hbm, dst_hbm,
               s_out, ex0_out, ex1_out,
               asrc_v, adst_v, si, di, sv, ex0_v, ex1_v):
    w = _wid()
    pltpu.sync_copy(asrc_hbm, asrc_v)
    pltpu.sync_copy(adst_hbm, adst_v)
    pltpu.sync_copy(src_hbm.at[w], si)
    pltpu.sync_copy(dst_hbm.at[w], di)

    zero = jnp.zeros((L,), jnp.float32)

    def zbody(i, _):
        sv[pl.ds(i * L, L)] = zero
        return 0
    lax.fori_loop(0, 2 * N // L, zbody, 0)

    def body(i, _):
        s16 = si[pl.ds(i * L, L)]
        d16 = di[pl.ds(i * L, L)]
        s2 = s16 * 2
        d2 = d16 * 2
        as0 = plsc.load_gather(asrc_v, [s2])
        as1 = plsc.load_gather(asrc_v, [s2 + 1])
        ad0 = plsc.load_gather(adst_v, [d2])
        ad1 = plsc.load_gather(adst_v, [d2 + 1])
        cs0 = plsc.load_gather(asrc_v, [d2])
        cs1 = plsc.load_gather(asrc_v, [d2 + 1])

        z0 = as0 + ad0
        e0 = jnp.maximum(z0, 0.2 * z0)
        zc0 = cs0 + ad0
        c0 = jnp.maximum(zc0, 0.2 * zc0)
        ex0 = jnp.exp(e0 - c0)

        z1 = as1 + ad1
        e1 = jnp.maximum(z1, 0.2 * z1)
        zc1 = cs1 + ad1
        c1 = jnp.maximum(zc1, 0.2 * zc1)
        ex1 = jnp.exp(e1 - c1)

        ex0_v[pl.ds(i * L, L)] = ex0
        ex1_v[pl.ds(i * L, L)] = ex1

        plsc.addupdate_scatter(sv, [d2], ex0)
        plsc.addupdate_scatter(sv, [d2 + 1], ex1)
        return 0
    lax.fori_loop(0, EW // L, body, 0)

    pltpu.sync_copy(ex0_v, ex0_out.at[w])
    pltpu.sync_copy(ex1_v, ex1_out.at[w])
    pltpu.sync_copy(sv, s_out.at[w])


_gata_kernel = functools.partial(
    pl.kernel,
    out_type=[
        jax.ShapeDtypeStruct((NW, 2 * N), jnp.float32),
        jax.ShapeDtypeStruct((NW, EW), jnp.float32),
        jax.ShapeDtypeStruct((NW, EW), jnp.float32),
    ],
    mesh=_mesh(),
    compiler_params=pltpu.CompilerParams(needs_layout_passes=False),
    scratch_types=[
        pltpu.VMEM((2 * N,), jnp.float32),
        pltpu.VMEM((2 * N,), jnp.float32),
        pltpu.VMEM((EW,), jnp.int32),
        pltpu.VMEM((EW,), jnp.int32),
        pltpu.VMEM((2 * N,), jnp.float32),
        pltpu.VMEM((EW,), jnp.float32),
        pltpu.VMEM((EW,), jnp.float32),
    ],
)(_gata_body)


def _ww_body(w_hbm, dst_hbm, ex0_hbm, ex1_hbm, ww0_out, ww1_out,
             wv, di, ex0_v, ex1_v):
    w = _wid()
    pltpu.sync_copy(w_hbm, wv)
    pltpu.sync_copy(dst_hbm.at[w], di)
    pltpu.sync_copy(ex0_hbm.at[w], ex0_v)
    pltpu.sync_copy(ex1_hbm.at[w], ex1_v)

    def body(i, _):
        d2 = di[pl.ds(i * L, L)] * 2
        w0 = plsc.load_gather(wv, [d2])
        w1 = plsc.load_gather(wv, [d2 + 1])
        ex0_v[pl.ds(i * L, L)] = ex0_v[pl.ds(i * L, L)] * w0
        ex1_v[pl.ds(i * L, L)] = ex1_v[pl.ds(i * L, L)] * w1
        return 0
    lax.fori_loop(0, EW // L, body, 0)

    pltpu.sync_copy(ex0_v, ww0_out.at[w])
    pltpu.sync_copy(ex1_v, ww1_out.at[w])


_ww_kernel = functools.partial(
    pl.kernel,
    out_type=[
        jax.ShapeDtypeStruct((NW, EW), jnp.float32),
        jax.ShapeDtypeStruct((NW, EW), jnp.float32),
    ],
    mesh=_mesh(),
    compiler_params=pltpu.CompilerParams(needs_layout_passes=False),
    scratch_types=[
        pltpu.VMEM((2 * N,), jnp.float32),
        pltpu.VMEM((EW,), jnp.int32),
        pltpu.VMEM((EW,), jnp.float32),
        pltpu.VMEM((EW,), jnp.float32),
    ],
)(_ww_body)


def _gatb_body(hg0_hbm, hg1_hbm, src_hbm, dst_hbm, ww0_hbm, ww1_hbm,
               zeros_hbm, out_hbm,
               srcblk, dstblk, w0blk, w1blk, bufa, bufb, semi, semg, acc):
    c = lax.axis_index("c")
    s = lax.axis_index("s")
    w = _wid()
    rs = s * RPT
    pltpu.sync_copy(zeros_hbm.at[pl.ds(rs, RPT)], acc.at[pl.ds(rs, RPT)])

    @pl.when(s == NS - 1)
    def _():
        pltpu.sync_copy(zeros_hbm.at[pl.ds(RTAIL, RREM)],
                        acc.at[pl.ds(RTAIL, RREM)])
    plsc.subcore_barrier()

    def issue_blk(b):
        sl = jnp.bitwise_and(b, 1)
        pltpu.async_copy(src_hbm.at[w * NBLK + b], srcblk.at[sl], semi)
        pltpu.async_copy(dst_hbm.at[w * NBLK + b], dstblk.at[sl], semi)
        pltpu.async_copy(ww0_hbm.at[w * NBLK + b], w0blk.at[sl], semi)
        pltpu.async_copy(ww1_hbm.at[w * NBLK + b], w1blk.at[sl], semi)

    def wait_blk(b):
        sl = jnp.bitwise_and(b, 1)
        pltpu.make_async_copy(src_hbm.at[w * NBLK + b], srcblk.at[sl],
                              semi).wait()
        pltpu.make_async_copy(dst_hbm.at[w * NBLK + b], dstblk.at[sl],
                              semi).wait()
        pltpu.make_async_copy(ww0_hbm.at[w * NBLK + b], w0blk.at[sl],
                              semi).wait()
        pltpu.make_async_copy(ww1_hbm.at[w * NBLK + b], w1blk.at[sl],
                              semi).wait()

    issue_blk(0)

    def chunk(cc, _):
        bsl = jnp.bitwise_and(lax.shift_right_logical(cc, 3), 1)
        r = jnp.bitwise_and(cc, 7)

        @pl.when(r == 0)
        def _():
            b = lax.shift_right_logical(cc, 3)
            wait_blk(b)

            @pl.when(b + 1 < NBLK)
            def _():
                issue_blk(b + 1)

        idx = srcblk.at[bsl, r]
        pltpu.async_copy(hg0_hbm.at[idx], bufa, semg)
        pltpu.async_copy(hg1_hbm.at[idx], bufb, semg)
        pltpu.make_async_copy(hg0_hbm.at[idx], bufa, semg).wait()
        pltpu.make_async_copy(hg1_hbm.at[idx], bufb, semg).wait()

        def ebody(e, _):
            bs = jnp.full((L,), bsl, jnp.int32)
            rr = jnp.full((L,), r, jnp.int32)
            es = jnp.full((L,), e, jnp.int32)
            b0 = plsc.load_gather(w0blk, [bs, rr, es])
            b1 = plsc.load_gather(w1blk, [bs, rr, es])
            for fb in range(HID // L):
                h0 = bufa[e, pl.ds(fb * L, L)]
                h1 = bufb[e, pl.ds(fb * L, L)]
                bufa[e, pl.ds(fb * L, L)] = b0 * h0 + b1 * h1
            return 0
        lax.fori_loop(0, K, ebody, 0)

        didx = dstblk.at[bsl, r]
        pltpu.sync_copy(bufa, acc.at[didx], add=True)
        return 0

    lax.fori_loop(0, NCH, chunk, 0)

    plsc.subcore_barrier()
    pltpu.sync_copy(acc.at[pl.ds(rs, RPT)],
                    out_hbm.at[pl.ds(c * NE + rs, RPT)])

    @pl.when(s == NS - 1)
    def _():
        pltpu.sync_copy(acc.at[pl.ds(RTAIL, RREM)],
                        out_hbm.at[pl.ds(c * NE + RTAIL, RREM)])


_gatb_kernel = functools.partial(
    pl.kernel,
    out_type=jax.ShapeDtypeStruct((NC * NE, HID), jnp.float32),
    mesh=_mesh(),
    compiler_params=pltpu.CompilerParams(needs_layout_passes=False),
    scratch_types=[
        pltpu.VMEM((2, 8, K), jnp.int32),
        pltpu.VMEM((2, 8, K), jnp.int32),
        pltpu.VMEM((2, 8, K), jnp.float32),
        pltpu.VMEM((2, 8, K), jnp.float32),
        pltpu.VMEM((K, HID), jnp.float32),
        pltpu.VMEM((K, HID), jnp.float32),
        pltpu.SemaphoreType.DMA,
        pltpu.SemaphoreType.DMA,
        pltpu.VMEM_SHARED((NE, HID), jnp.float32),
    ],
)(_gatb_body)


def _dis_from(degp_ref):
    deg = jnp.sum(degp_ref[...], axis=0) + 1.0
    return lax.rsqrt(deg)


def _tc1_body(degp_ref, x_ref, w1_ref, hs1_ref):
    dis = _dis_from(degp_ref)
    h = jnp.dot(x_ref[...], w1_ref[...], preferred_element_type=jnp.float32)
    hs1_ref[0:N, :] = h * dis[:, None]
    hs1_ref[N:NE, :] = jnp.zeros((NE - N, HID), jnp.float32)


def _tc2_body(degp_ref, acc_ref, hs_ref, b_ref, w2_ref, hs2_ref):
    dis = _dis_from(degp_ref)
    a = acc_ref[0:N, :] + acc_ref[NE:NE + N, :] + hs_ref[0:N, :]
    h = jax.nn.relu(a * dis[:, None] + b_ref[...])
    h2 = jnp.dot(h, w2_ref[...], preferred_element_type=jnp.float32)
    hs2_ref[0:N, :] = h2 * dis[:, None]
    hs2_ref[N:NE, :] = jnp.zeros((NE - N, HID), jnp.float32)


def _tc3_body(degp_ref, acc_ref, hs_ref, b_ref, wg_ref, ats_ref, atd_ref,
              hg0_ref, hg1_ref, asrc_ref, adst_ref):
    dis = _dis_from(degp_ref)
    a = acc_ref[0:N, :] + acc_ref[NE:NE + N, :] + hs_ref[0:N, :]
    h = jax.nn.relu(a * dis[:, None] + b_ref[...])
    hg = jnp.dot(h, wg_ref[...], preferred_element_type=jnp.float32)
    hg0_ref[0:N, :] = hg[:, 0:HID]
    hg0_ref[N:NE, :] = jnp.zeros((NE - N, HID), jnp.float32)
    hg1_ref[0:N, :] = hg[:, HID:2 * HID]
    hg1_ref[N:NE, :] = jnp.zeros((NE - N, HID), jnp.float32)
    a0 = jnp.dot(hg[:, 0:HID], ats_ref[0, :],
                 preferred_element_type=jnp.float32)
    a1 = jnp.dot(hg[:, HID:2 * HID], ats_ref[1, :],
                 preferred_element_type=jnp.float32)
    asrc_ref[...] = jnp.stack([a0, a1], axis=1)
    d0 = jnp.dot(hg[:, 0:HID], atd_ref[0, :],
                 preferred_element_type=jnp.float32)
    d1 = jnp.dot(hg[:, HID:2 * HID], atd_ref[1, :],
                 preferred_element_type=jnp.float32)
    adst_ref[...] = jnp.stack([d0, d1], axis=1)


def _tc4_body(s_ref, w_ref):
    w_ref[...] = 0.5 / (jnp.sum(s_ref[...], axis=0) + 1.0)


def _tc5_body(acc_ref, hg0_ref, hg1_ref, w_ref, bg_ref, batch_ref,
              wl1_ref, bl1_ref, wl2_ref, bl2_ref, o_ref, mx_ref):
    wgt = w_ref[...]
    h = (acc_ref[0:N, :] + acc_ref[NE:NE + N, :]
         + wgt[:, 0:1] * hg0_ref[0:N, :] + wgt[:, 1:2] * hg1_ref[0:N, :])
    h = jax.nn.relu(h + bg_ref[...])

    batch = batch_ref[...]
    gids = lax.broadcasted_iota(jnp.int32, (N, NGRAPH), 1)
    onehot = (batch[:, None] == gids).astype(jnp.float32)
    counts = jnp.sum(onehot, axis=0)
    meansum = lax.dot_general(onehot, h, (((0,), (0,)), ((), ())),
                              preferred_element_type=jnp.float32)
    mean = meansum / jnp.maximum(counts, 1.0)[:, None]

    def body(g, _):
        m = jnp.where(batch[:, None] == g, h, -jnp.inf)
        row = jnp.max(m, axis=0)
        row = jnp.where(jnp.isfinite(row), row, 0.0)
        mx_ref[pl.ds(g, 1), :] = row[None, :]
        return 0
    lax.fori_loop(0, NGRAPH, body, 0)

    g = mean + mx_ref[...]
    g = jax.nn.relu(jnp.dot(g, wl1_ref[...],
                            preferred_element_type=jnp.float32) + bl1_ref[...])
    o_ref[...] = jnp.dot(g, wl2_ref[...],
                         preferred_element_type=jnp.float32) + bl2_ref[...]


def kernel(x, edge_index, batch, W1, b1, W2, b2, Wg, att_src, att_dst, bg,
           Wl1, bl1, Wl2, bl2):
    src = edge_index[0]
    dst = edge_index[1]
    src_w = src.reshape(NW, EW)
    dst_w = dst.reshape(NW, EW)
    padi = jnp.full((NW, EWP - EW), N, jnp.int32)
    src_p = jnp.concatenate([src_w, padi], axis=1).reshape(NW * NBLK, 8, K)
    dst_p = jnp.concatenate([dst_w, padi], axis=1).reshape(NW * NBLK, 8, K)
    zeros_pool = jnp.zeros((NE, HID), jnp.float32)

    degp = _deg_kernel(dst_w)

    hs1 = pl.pallas_call(
        _tc1_body,
        out_shape=jax.ShapeDtypeStruct((NE, HID), jnp.float32),
    )(degp, x, W1)

    acc1 = _gcn_kernel(hs1, src_p, dst_p, zeros_pool)

    hs2 = pl.pallas_call(
        _tc2_body,
        out_shape=jax.ShapeDtypeStruct((NE, HID), jnp.float32),
    )(degp, acc1, hs1, b1, W2)

    acc2 = _gcn_kernel(hs2, src_p, dst_p, zeros_pool)

    hg0, hg1, asrc, adst = pl.pallas_call(
        _tc3_body,
        out_shape=[
            jax.ShapeDtypeStruct((NE, HID), jnp.float32),
            jax.ShapeDtypeStruct((NE, HID), jnp.float32),
            jax.ShapeDtypeStruct((N, HEADS), jnp.float32),
            jax.ShapeDtypeStruct((N, HEADS), jnp.float32),
        ],
    )(degp, acc2, hs2, b2, Wg, att_src, att_dst)

    s_part, ex0, ex1 = _gata_kernel(asrc.reshape(2 * N), adst.reshape(2 * N),
                                    src_w, dst_w)

    wtab = pl.pallas_call(
        _tc4_body,
        out_shape=jax.ShapeDtypeStruct((2 * N,), jnp.float32),
    )(s_part)

    ww0, ww1 = _ww_kernel(wtab, dst_w, ex0, ex1)
    padw = jnp.zeros((NW, EWP - EW), jnp.float32)
    ww0p = jnp.concatenate([ww0, padw], axis=1).reshape(NW * NBLK, 8, K)
    ww1p = jnp.concatenate([ww1, padw], axis=1).reshape(NW * NBLK, 8, K)

    accg = _gatb_kernel(hg0, hg1, src_p, dst_p, ww0p, ww1p, zeros_pool)

    return pl.pallas_call(
        _tc5_body,
        out_shape=jax.ShapeDtypeStruct((NGRAPH, NCLS), jnp.float32),
        scratch_shapes=[pltpu.VMEM((NGRAPH, HID), jnp.float32)],
    )(accg, hg0, hg1, wtab.reshape(N, HEADS), bg, batch, Wl1, bl1, Wl2, bl2)


# speedup vs baseline: 26.9645x; 1.1181x over previous
import functools

import jax
import jax.numpy as jnp
from jax import lax
from jax.experimental import pallas as pl
from jax.experimental.pallas import tpu as pltpu
from jax.experimental.pallas import tpu_sc as plsc

N = 10000
E = 320000
F_IN = 128
HID = 128
HEADS = 2
NCLS = 2
NGRAPH = 64

NC = 2
NS = 16
NW = NC * NS
L = 16

EW = E // NW
K = 128
NCH = 80
NBLK = NCH // 8
EWP = NCH * K

KB = 64
CPB = 16
NCHB = EWP // KB

NE = N + 8
RPT = 624
RTAIL = NS * RPT
RREM = NE - RTAIL


def _mesh():
    return plsc.VectorSubcoreMesh(core_axis_name="c", subcore_axis_name="s")


def _wid():
    return lax.axis_index("s") * NC + lax.axis_index("c")


def _deg_body(dst_hbm, out_hbm, deg_v, idx_v):
    w = _wid()
    zero = jnp.zeros((L,), jnp.float32)

    def zbody(i, _):
        deg_v[pl.ds(i * L, L)] = zero
        return 0
    lax.fori_loop(0, N // L, zbody, 0)

    pltpu.sync_copy(dst_hbm.at[w], idx_v)
    ones = jnp.ones((L,), jnp.float32)

    def body(i, _):
        d = idx_v[pl.ds(i * L, L)]
        plsc.addupdate_scatter(deg_v, [d], ones)
        return 0
    lax.fori_loop(0, EW // L, body, 0)

    pltpu.sync_copy(deg_v, out_hbm.at[w])


_deg_kernel = functools.partial(
    pl.kernel,
    out_type=jax.ShapeDtypeStruct((NW, N), jnp.float32),
    mesh=_mesh(),
    compiler_params=pltpu.CompilerParams(needs_layout_passes=False),
    scratch_types=[
        pltpu.VMEM((N,), jnp.float32),
        pltpu.VMEM((EW,), jnp.int32),
    ],
)(_deg_body)


def _gcn_body(hs_hbm, src_hbm, dst_hbm, zeros_hbm, out_hbm,
              srcblk, dstblk, buf, semi, semg0, semg1, acc):
    c = lax.axis_index("c")
    s = lax.axis_index("s")
    w = _wid()
    rs = s * RPT
    pltpu.sync_copy(zeros_hbm.at[pl.ds(rs, RPT)], acc.at[pl.ds(rs, RPT)])

    @pl.when(s == NS - 1)
    def _():
        pltpu.sync_copy(zeros_hbm.at[pl.ds(RTAIL, RREM)],
                        acc.at[pl.ds(RTAIL, RREM)])
    plsc.subcore_barrier()

    semg = (semg0, semg1)

    def issue_blk(b):
        sl = jnp.bitwise_and(b, 1)
        pltpu.async_copy(src_hbm.at[w * NBLK + b], srcblk.at[sl], semi)
        pltpu.async_copy(dst_hbm.at[w * NBLK + b], dstblk.at[sl], semi)

    def wait_blk(b):
        sl = jnp.bitwise_and(b, 1)
        pltpu.make_async_copy(src_hbm.at[w * NBLK + b], srcblk.at[sl],
                              semi).wait()
        pltpu.make_async_copy(dst_hbm.at[w * NBLK + b], dstblk.at[sl],
                              semi).wait()

    def issue_gather(cc, p):
        idx = srcblk.at[jnp.bitwise_and(lax.shift_right_logical(cc, 3), 1),
                        jnp.bitwise_and(cc, 7)]
        pltpu.async_copy(hs_hbm.at[idx], buf.at[p], semg[p])

    def wait_gather(cc, p):
        idx = srcblk.at[jnp.bitwise_and(lax.shift_right_logical(cc, 3), 1),
                        jnp.bitwise_and(cc, 7)]
        pltpu.make_async_copy(hs_hbm.at[idx], buf.at[p], semg[p]).wait()

    issue_blk(0)
    wait_blk(0)
    issue_blk(1)
    issue_gather(0, 0)

    def chunk(cc, p):
        c1 = cc + 1

        @pl.when(c1 < NCH)
        def _():
            @pl.when(jnp.bitwise_and(c1, 7) == 0)
            def _():
                wait_blk(lax.shift_right_logical(c1, 3))
            issue_gather(c1, 1 - p)

        wait_gather(cc, p)
        didx = dstblk.at[jnp.bitwise_and(lax.shift_right_logical(cc, 3), 1),
                         jnp.bitwise_and(cc, 7)]
        pltpu.sync_copy(buf.at[p], acc.at[didx], add=True)

        @pl.when(jnp.logical_and(jnp.bitwise_and(c1, 7) == 0,
                                 c1 + 8 < NCH))
        def _():
            issue_blk(lax.shift_right_logical(c1, 3) + 1)

    def super_body(t, _):
        chunk(2 * t, 0)
        chunk(2 * t + 1, 1)
        return 0
    lax.fori_loop(0, NCH // 2, super_body, 0)

    plsc.subcore_barrier()
    pltpu.sync_copy(acc.at[pl.ds(rs, RPT)],
                    out_hbm.at[pl.ds(c * NE + rs, RPT)])

    @pl.when(s == NS - 1)
    def _():
        pltpu.sync_copy(acc.at[pl.ds(RTAIL, RREM)],
                        out_hbm.at[pl.ds(c * NE + RTAIL, RREM)])


_gcn_kernel = functools.partial(
    pl.kernel,
    out_type=jax.ShapeDtypeStruct((NC * NE, HID), jnp.float32),
    mesh=_mesh(),
    compiler_params=pltpu.CompilerParams(needs_layout_passes=False),
    scratch_types=[
        pltpu.VMEM((2, 8, K), jnp.int32),
        pltpu.VMEM((2, 8, K), jnp.int32),
        pltpu.VMEM((2, K, HID), jnp.float32),
        pltpu.SemaphoreType.DMA,
        pltpu.SemaphoreType.DMA,
        pltpu.SemaphoreType.DMA,
        pltpu.VMEM_SHARED((NE, HID), jnp.float32),
    ],
)(_gcn_body)


def _gata_body(asrc_hbm, adst_hbm, src_hbm, dst_hbm,
               s_out, ex0_out, ex1_out,
               asrc_v, adst_v, si, di, sv, ex0_v, ex1_v):
    w = _wid()
    pltpu.sync_copy(asrc_hbm, asrc_v)
    pltpu.sync_copy(adst_hbm, adst_v)
    pltpu.sync_copy(src_hbm.at[w], si)
    pltpu.sync_copy(dst_hbm.at[w], di)

    zero = jnp.zeros((L,), jnp.float32)

    def zbody(i, _):
        sv[pl.ds(i * L, L)] = zero
        return 0
    lax.fori_loop(0, 2 * N // L, zbody, 0)

    def body(i, _):
        s16 = si[pl.ds(i * L, L)]
        d16 = di[pl.ds(i * L, L)]
        s2 = s16 * 2
        d2 = d16 * 2
        as0 = plsc.load_gather(asrc_v, [s2])
        as1 = plsc.load_gather(asrc_v, [s2 + 1])
        ad0 = plsc.load_gather(adst_v, [d2])
        ad1 = plsc.load_gather(adst_v, [d2 + 1])
        cs0 = plsc.load_gather(asrc_v, [d2])
        cs1 = plsc.load_gather(asrc_v, [d2 + 1])

        z0 = as0 + ad0
        e0 = jnp.maximum(z0, 0.2 * z0)
        zc0 = cs0 + ad0
        c0 = jnp.maximum(zc0, 0.2 * zc0)
        ex0 = jnp.exp(e0 - c0)

        z1 = as1 + ad1
        e1 = jnp.maximum(z1, 0.2 * z1)
        zc1 = cs1 + ad1
        c1 = jnp.maximum(zc1, 0.2 * zc1)
        ex1 = jnp.exp(e1 - c1)

        ex0_v[pl.ds(i * L, L)] = ex0
        ex1_v[pl.ds(i * L, L)] = ex1

        plsc.addupdate_scatter(sv, [d2], ex0)
        plsc.addupdate_scatter(sv, [d2 + 1], ex1)
        return 0
    lax.fori_loop(0, EW // L, body, 0)

    pltpu.sync_copy(ex0_v, ex0_out.at[w])
    pltpu.sync_copy(ex1_v, ex1_out.at[w])
    pltpu.sync_copy(sv, s_out.at[w])


_gata_kernel = functools.partial(
    pl.kernel,
    out_type=[
        jax.ShapeDtypeStruct((NW, 2 * N), jnp.float32),
        jax.ShapeDtypeStruct((NW, EW), jnp.float32),
        jax.ShapeDtypeStruct((NW, EW), jnp.float32),
    ],
    mesh=_mesh(),
    compiler_params=pltpu.CompilerParams(needs_layout_passes=False),
    scratch_types=[
        pltpu.VMEM((2 * N,), jnp.float32),
        pltpu.VMEM((2 * N,), jnp.float32),
        pltpu.VMEM((EW,), jnp.int32),
        pltpu.VMEM((EW,), jnp.int32),
        pltpu.VMEM((2 * N,), jnp.float32),
        pltpu.VMEM((EW,), jnp.float32),
        pltpu.VMEM((EW,), jnp.float32),
    ],
)(_gata_body)


def _ww_body(w_hbm, dst_hbm, ex0_hbm, ex1_hbm, ww0_out, ww1_out,
             wv, di, ex0_v, ex1_v):
    w = _wid()
    pltpu.sync_copy(w_hbm, wv)
    pltpu.sync_copy(dst_hbm.at[w], di)
    pltpu.sync_copy(ex0_hbm.at[w], ex0_v)
    pltpu.sync_copy(ex1_hbm.at[w], ex1_v)

    def body(i, _):
        d2 = di[pl.ds(i * L, L)] * 2
        w0 = plsc.load_gather(wv, [d2])
        w1 = plsc.load_gather(wv, [d2 + 1])
        ex0_v[pl.ds(i * L, L)] = ex0_v[pl.ds(i * L, L)] * w0
        ex1_v[pl.ds(i * L, L)] = ex1_v[pl.ds(i * L, L)] * w1
        return 0
    lax.fori_loop(0, EW // L, body, 0)

    pltpu.sync_copy(ex0_v, ww0_out.at[w])
    pltpu.sync_copy(ex1_v, ww1_out.at[w])


_ww_kernel = functools.partial(
    pl.kernel,
    out_type=[
        jax.ShapeDtypeStruct((NW, EW), jnp.float32),
        jax.ShapeDtypeStruct((NW, EW), jnp.float32),
    ],
    mesh=_mesh(),
    compiler_params=pltpu.CompilerParams(needs_layout_passes=False),
    scratch_types=[
        pltpu.VMEM((2 * N,), jnp.float32),
        pltpu.VMEM((EW,), jnp.int32),
        pltpu.VMEM((EW,), jnp.float32),
        pltpu.VMEM((EW,), jnp.float32),
    ],
)(_ww_body)


def _gatb_body(hg0_hbm, hg1_hbm, src_hbm, dst_hbm, ww0_hbm, ww1_hbm,
               zeros_hbm, out_hbm,
               srcblk, dstblk, w0blk, w1blk, bufa, bufb,
               semi, semg0, semg1, acc):
    c = lax.axis_index("c")
    s = lax.axis_index("s")
    w = _wid()
    rs = s * RPT
    pltpu.sync_copy(zeros_hbm.at[pl.ds(rs, RPT)], acc.at[pl.ds(rs, RPT)])

    @pl.when(s == NS - 1)
    def _():
        pltpu.sync_copy(zeros_hbm.at[pl.ds(RTAIL, RREM)],
                        acc.at[pl.ds(RTAIL, RREM)])
    plsc.subcore_barrier()

    semg = (semg0, semg1)

    def issue_blk(b):
        sl = jnp.bitwise_and(b, 1)
        pltpu.async_copy(src_hbm.at[w * NBLK + b], srcblk.at[sl], semi)
        pltpu.async_copy(dst_hbm.at[w * NBLK + b], dstblk.at[sl], semi)
        pltpu.async_copy(ww0_hbm.at[w * NBLK + b], w0blk.at[sl], semi)
        pltpu.async_copy(ww1_hbm.at[w * NBLK + b], w1blk.at[sl], semi)

    def wait_blk(b):
        sl = jnp.bitwise_and(b, 1)
        pltpu.make_async_copy(src_hbm.at[w * NBLK + b], srcblk.at[sl],
                              semi).wait()
        pltpu.make_async_copy(dst_hbm.at[w * NBLK + b], dstblk.at[sl],
                              semi).wait()
        pltpu.make_async_copy(ww0_hbm.at[w * NBLK + b], w0blk.at[sl],
                              semi).wait()
        pltpu.make_async_copy(ww1_hbm.at[w * NBLK + b], w1blk.at[sl],
                              semi).wait()

    def issue_gather(cc, p):
        idx = srcblk.at[jnp.bitwise_and(lax.shift_right_logical(cc, 4), 1),
                        jnp.bitwise_and(cc, 15)]
        pltpu.async_copy(hg0_hbm.at[idx], bufa.at[p], semg[p])
        pltpu.async_copy(hg1_hbm.at[idx], bufb.at[p], semg[p])

    def wait_gather(cc, p):
        idx = srcblk.at[jnp.bitwise_and(lax.shift_right_logical(cc, 4), 1),
                        jnp.bitwise_and(cc, 15)]
        pltpu.make_async_copy(hg0_hbm.at[idx], bufa.at[p], semg[p]).wait()
        pltpu.make_async_copy(hg1_hbm.at[idx], bufb.at[p], semg[p]).wait()

    issue_blk(0)
    wait_blk(0)
    issue_blk(1)
    issue_gather(0, 0)

    def chunk(cc, p):
        c1 = cc + 1
        bsl = jnp.bitwise_and(lax.shift_right_logical(cc, 4), 1)
        r = jnp.bitwise_and(cc, 15)

        @pl.when(c1 < NCHB)
        def _():
            @pl.when(jnp.bitwise_and(c1, 15) == 0)
            def _():
                wait_blk(lax.shift_right_logical(c1, 4))
            issue_gather(c1, 1 - p)

        wait_gather(cc, p)

        def ebody(e, _):
            bs = jnp.full((L,), bsl, jnp.int32)
            rr = jnp.full((L,), r, jnp.int32)
            es = jnp.full((L,), e, jnp.int32)
            b0 = plsc.load_gather(w0blk, [bs, rr, es])
            b1 = plsc.load_gather(w1blk, [bs, rr, es])
            for fb in range(HID // L):
                h0 = bufa[p, e, pl.ds(fb * L, L)]
                h1 = bufb[p, e, pl.ds(fb * L, L)]
                bufa[p, e, pl.ds(fb * L, L)] = b0 * h0 + b1 * h1
            return 0
        lax.fori_loop(0, KB, ebody, 0)

        didx = dstblk.at[bsl, r]
        pltpu.sync_copy(bufa.at[p], acc.at[didx], add=True)

        @pl.when(jnp.logical_and(jnp.bitwise_and(c1, 15) == 0,
                                 c1 + CPB < NCHB))
        def _():
            issue_blk(lax.shift_right_logical(c1, 4) + 1)

    def super_body(t, _):
        chunk(2 * t, 0)
        chunk(2 * t + 1, 1)
        return 0
    lax.fori_loop(0, NCHB // 2, super_body, 0)

    plsc.subcore_barrier()
    pltpu.sync_copy(acc.at[pl.ds(rs, RPT)],
                    out_hbm.at[pl.ds(c * NE + rs, RPT)])

    @pl.when(s == NS - 1)
    def _():
        pltpu.sync_copy(acc.at[pl.ds(RTAIL, RREM)],
                        out_hbm.at[pl.ds(c * NE + RTAIL, RREM)])


_gatb_kernel = functools.partial(
    pl.kernel,
    out_type=jax.ShapeDtypeStruct((NC * NE, HID), jnp.float32),
    mesh=_mesh(),
    compiler_params=pltpu.CompilerParams(needs_layout_passes=False),
    scratch_types=[
        pltpu.VMEM((2, CPB, KB), jnp.int32),
        pltpu.VMEM((2, CPB, KB), jnp.int32),
        pltpu.VMEM((2, CPB, KB), jnp.float32),
        pltpu.VMEM((2, CPB, KB), jnp.float32),
        pltpu.VMEM((2, KB, HID), jnp.float32),
        pltpu.VMEM((2, KB, HID), jnp.float32),
        pltpu.SemaphoreType.DMA,
        pltpu.SemaphoreType.DMA,
        pltpu.SemaphoreType.DMA,
        pltpu.VMEM_SHARED((NE, HID), jnp.float32),
    ],
)(_gatb_body)


def _dis_from(degp_ref):
    deg = jnp.sum(degp_ref[...], axis=0) + 1.0
    return lax.rsqrt(deg)


def _tc1_body(degp_ref, x_ref, w1_ref, hs1_ref):
    dis = _dis_from(degp_ref)
    h = jnp.dot(x_ref[...], w1_ref[...], preferred_element_type=jnp.float32)
    hs1_ref[0:N, :] = h * dis[:, None]
    hs1_ref[N:NE, :] = jnp.zeros((NE - N, HID), jnp.float32)


def _tc2_body(degp_ref, acc_ref, hs_ref, b_ref, w2_ref, hs2_ref):
    dis = _dis_from(degp_ref)
    a = acc_ref[0:N, :] + acc_ref[NE:NE + N, :] + hs_ref[0:N, :]
    h = jax.nn.relu(a * dis[:, None] + b_ref[...])
    h2 = jnp.dot(h, w2_ref[...], preferred_element_type=jnp.float32)
    hs2_ref[0:N, :] = h2 * dis[:, None]
    hs2_ref[N:NE, :] = jnp.zeros((NE - N, HID), jnp.float32)


def _tc3_body(degp_ref, acc_ref, hs_ref, b_ref, wg_ref, ats_ref, atd_ref,
              hg0_ref, hg1_ref, asrc_ref, adst_ref):
    dis = _dis_from(degp_ref)
    a = acc_ref[0:N, :] + acc_ref[NE:NE + N, :] + hs_ref[0:N, :]
    h = jax.nn.relu(a * dis[:, None] + b_ref[...])
    hg = jnp.dot(h, wg_ref[...], preferred_element_type=jnp.float32)
    hg0_ref[0:N, :] = hg[:, 0:HID]
    hg0_ref[N:NE, :] = jnp.zeros((NE - N, HID), jnp.float32)
    hg1_ref[0:N, :] = hg[:, HID:2 * HID]
    hg1_ref[N:NE, :] = jnp.zeros((NE - N, HID), jnp.float32)
    a0 = jnp.dot(hg[:, 0:HID], ats_ref[0, :],
                 preferred_element_type=jnp.float32)
    a1 = jnp.dot(hg[:, HID:2 * HID], ats_ref[1, :],
                 preferred_element_type=jnp.float32)
    asrc_ref[...] = jnp.stack([a0, a1], axis=1)
    d0 = jnp.dot(hg[:, 0:HID], atd_ref[0, :],
                 preferred_element_type=jnp.float32)
    d1 = jnp.dot(hg[:, HID:2 * HID], atd_ref[1, :],
                 preferred_element_type=jnp.float32)
    adst_ref[...] = jnp.stack([d0, d1], axis=1)


def _tc4_body(s_ref, w_ref):
    w_ref[...] = 0.5 / (jnp.sum(s_ref[...], axis=0) + 1.0)


def _tc5_body(acc_ref, hg0_ref, hg1_ref, w_ref, bg_ref, batch_ref,
              wl1_ref, bl1_ref, wl2_ref, bl2_ref, o_ref, mx_ref):
    wgt = w_ref[...]
    h = (acc_ref[0:N, :] + acc_ref[NE:NE + N, :]
         + wgt[:, 0:1] * hg0_ref[0:N, :] + wgt[:, 1:2] * hg1_ref[0:N, :])
    h = jax.nn.relu(h + bg_ref[...])

    batch = batch_ref[...]
    gids = lax.broadcasted_iota(jnp.int32, (N, NGRAPH), 1)
    onehot = (batch[:, None] == gids).astype(jnp.float32)
    counts = jnp.sum(onehot, axis=0)
    meansum = lax.dot_general(onehot, h, (((0,), (0,)), ((), ())),
                              preferred_element_type=jnp.float32)
    mean = meansum / jnp.maximum(counts, 1.0)[:, None]

    def body(g, _):
        m = jnp.where(batch[:, None] == g, h, -jnp.inf)
        row = jnp.max(m, axis=0)
        row = jnp.where(jnp.isfinite(row), row, 0.0)
        mx_ref[pl.ds(g, 1), :] = row[None, :]
        return 0
    lax.fori_loop(0, NGRAPH, body, 0)

    g = mean + mx_ref[...]
    g = jax.nn.relu(jnp.dot(g, wl1_ref[...],
                            preferred_element_type=jnp.float32) + bl1_ref[...])
    o_ref[...] = jnp.dot(g, wl2_ref[...],
                         preferred_element_type=jnp.float32) + bl2_ref[...]


def kernel(x, edge_index, batch, W1, b1, W2, b2, Wg, att_src, att_dst, bg,
           Wl1, bl1, Wl2, bl2):
    src = edge_index[0]
    dst = edge_index[1]
    src_w = src.reshape(NW, EW)
    dst_w = dst.reshape(NW, EW)
    padi = jnp.full((NW, EWP - EW), N, jnp.int32)
    src_p = jnp.concatenate([src_w, padi], axis=1).reshape(NW * NBLK, 8, K)
    dst_p = jnp.concatenate([dst_w, padi], axis=1).reshape(NW * NBLK, 8, K)
    zeros_pool = jnp.zeros((NE, HID), jnp.float32)

    degp = _deg_kernel(dst_w)

    hs1 = pl.pallas_call(
        _tc1_body,
        out_shape=jax.ShapeDtypeStruct((NE, HID), jnp.float32),
    )(degp, x, W1)

    acc1 = _gcn_kernel(hs1, src_p, dst_p, zeros_pool)

    hs2 = pl.pallas_call(
        _tc2_body,
        out_shape=jax.ShapeDtypeStruct((NE, HID), jnp.float32),
    )(degp, acc1, hs1, b1, W2)

    acc2 = _gcn_kernel(hs2, src_p, dst_p, zeros_pool)

    hg0, hg1, asrc, adst = pl.pallas_call(
        _tc3_body,
        out_shape=[
            jax.ShapeDtypeStruct((NE, HID), jnp.float32),
            jax.ShapeDtypeStruct((NE, HID), jnp.float32),
            jax.ShapeDtypeStruct((N, HEADS), jnp.float32),
            jax.ShapeDtypeStruct((N, HEADS), jnp.float32),
        ],
    )(degp, acc2, hs2, b2, Wg, att_src, att_dst)

    s_part, ex0, ex1 = _gata_kernel(asrc.reshape(2 * N), adst.reshape(2 * N),
                                    src_w, dst_w)

    wtab = pl.pallas_call(
        _tc4_body,
        out_shape=jax.ShapeDtypeStruct((2 * N,), jnp.float32),
    )(s_part)

    ww0, ww1 = _ww_kernel(wtab, dst_w, ex0, ex1)
    padw = jnp.zeros((NW, EWP - EW), jnp.float32)
    ww0p = jnp.concatenate([ww0, padw], axis=1).reshape(NW * NBLK, CPB, KB)
    ww1p = jnp.concatenate([ww1, padw], axis=1).reshape(NW * NBLK, CPB, KB)
    src_p2 = src_p.reshape(NW * NBLK, CPB, KB)
    dst_p2 = dst_p.reshape(NW * NBLK, CPB, KB)

    accg = _gatb_kernel(hg0, hg1, src_p2, dst_p2, ww0p, ww1p, zeros_pool)

    return pl.pallas_call(
        _tc5_body,
        out_shape=jax.ShapeDtypeStruct((NGRAPH, NCLS), jnp.float32),
        scratch_shapes=[pltpu.VMEM((NGRAPH, HID), jnp.float32)],
    )(accg, hg0, hg1, wtab.reshape(N, HEADS), bg, batch, Wl1, bl1, Wl2, bl2)
